# baseline (speedup 1.0000x reference)
import functools

import jax
import jax.numpy as jnp
from jax import lax
from jax.experimental import pallas as pl
from jax.experimental.pallas import tpu as pltpu
from jax.experimental.pallas import tpu_sc as plsc

NC, NS, LANES = 2, 16, 16
NW = NC * NS
N_PAD = 10240
NPT = N_PAD // NW
NPS = N_PAD // NS
H = 32
CHUNK = 128
BM = 1024
_PREC = lax.Precision.HIGHEST


def _mesh():
    return plsc.VectorSubcoreMesh(core_axis_name="c", subcore_axis_name="s")


def _tc_mlp(x_pad, W1, b1, W2, b2, d0_col, d1_col):
    F = x_pad.shape[1]
    Hh = W1.shape[1]

    def body(x_ref, w1_ref, b1_ref, w2_ref, b2_ref, d0_ref, d1_ref,
             h_ref, q_ref, dinv_ref):
        h1 = jnp.dot(x_ref[...], w1_ref[...],
                     preferred_element_type=jnp.float32, precision=_PREC)
        h1 = jnp.maximum(h1 + b1_ref[...], 0.0)
        h2 = jnp.dot(h1, w2_ref[...],
                     preferred_element_type=jnp.float32, precision=_PREC)
        h = jnp.maximum(h2 + b2_ref[...], 0.0)
        h_ref[...] = h
        dinv = lax.rsqrt(jnp.maximum(d0_ref[...] + d1_ref[...], 1.0))
        dinv_ref[...] = dinv
        q_ref[...] = h * dinv

    return pl.pallas_call(
        body,
        grid=(N_PAD // BM,),
        in_specs=[
            pl.BlockSpec((BM, F), lambda i: (i, 0)),
            pl.BlockSpec((F, Hh), lambda i: (0, 0)),
            pl.BlockSpec((1, Hh), lambda i: (0, 0)),
            pl.BlockSpec((Hh, Hh), lambda i: (0, 0)),
            pl.BlockSpec((1, Hh), lambda i: (0, 0)),
            pl.BlockSpec((BM, 1), lambda i: (i, 0)),
            pl.BlockSpec((BM, 1), lambda i: (i, 0)),
        ],
        out_specs=[
            pl.BlockSpec((BM, Hh), lambda i: (i, 0)),
            pl.BlockSpec((BM, Hh), lambda i: (i, 0)),
            pl.BlockSpec((BM, 1), lambda i: (i, 0)),
        ],
        out_shape=[
            jax.ShapeDtypeStruct((N_PAD, Hh), jnp.float32),
            jax.ShapeDtypeStruct((N_PAD, Hh), jnp.float32),
            jax.ShapeDtypeStruct((N_PAD, 1), jnp.float32),
        ],
    )(x_pad, W1, b1, W2, b2, d0_col, d1_col)


def _tc_combine(h, a0, a1, dinv_col):
    def body(h_ref, a0_ref, a1_ref, dv_ref, p_ref, q_ref):
        dinv = dv_ref[...]
        p = h_ref[...] - dinv * (a0_ref[...] + a1_ref[...])
        p_ref[...] = p
        q_ref[...] = p * dinv

    return pl.pallas_call(
        body,
        grid=(N_PAD // BM,),
        in_specs=[
            pl.BlockSpec((BM, H), lambda i: (i, 0)),
            pl.BlockSpec((BM, H), lambda i: (i, 0)),
            pl.BlockSpec((BM, H), lambda i: (i, 0)),
            pl.BlockSpec((BM, 1), lambda i: (i, 0)),
        ],
        out_specs=[
            pl.BlockSpec((BM, H), lambda i: (i, 0)),
            pl.BlockSpec((BM, H), lambda i: (i, 0)),
        ],
        out_shape=[
            jax.ShapeDtypeStruct((N_PAD, H), jnp.float32),
            jax.ShapeDtypeStruct((N_PAD, H), jnp.float32),
        ],
    )(h, a0, a1, dinv_col)


def _tc_head(h, p1, b0, b1_agg, dinv_col, A0, A1, A2, bm1, Wm2, bm2):
    C = Wm2.shape[1]

    def body(h_ref, p1_ref, b0_ref, b1_ref, dv_ref, a0, a1, a2, b1r, w2r, b2r,
             o_ref):
        p1v = p1_ref[...]
        p2 = p1v - dv_ref[...] * (b0_ref[...] + b1_ref[...])
        t = jnp.dot(h_ref[...], a0[...],
                    preferred_element_type=jnp.float32, precision=_PREC)
        t += jnp.dot(p1v, a1[...],
                     preferred_element_type=jnp.float32, precision=_PREC)
        t += jnp.dot(p2, a2[...],
                     preferred_element_type=jnp.float32, precision=_PREC)
        z = jnp.maximum(t + b1r[...], 0.0)
        o_ref[...] = jnp.dot(z, w2r[...],
                             preferred_element_type=jnp.float32,
                             precision=_PREC) + b2r[...]

    return pl.pallas_call(
        body,
        grid=(N_PAD // BM,),
        in_specs=[
            pl.BlockSpec((BM, H), lambda i: (i, 0)),
            pl.BlockSpec((BM, H), lambda i: (i, 0)),
            pl.BlockSpec((BM, H), lambda i: (i, 0)),
            pl.BlockSpec((BM, H), lambda i: (i, 0)),
            pl.BlockSpec((BM, 1), lambda i: (i, 0)),
            pl.BlockSpec((H, H), lambda i: (0, 0)),
            pl.BlockSpec((H, H), lambda i: (0, 0)),
            pl.BlockSpec((H, H), lambda i: (0, 0)),
            pl.BlockSpec((1, H), lambda i: (0, 0)),
            pl.BlockSpec((H, C), lambda i: (0, 0)),
            pl.BlockSpec((1, C), lambda i: (0, 0)),
        ],
        out_specs=pl.BlockSpec((BM, C), lambda i: (i, 0)),
        out_shape=jax.ShapeDtypeStruct((N_PAD, C), jnp.float32),
    )(h, p1, b0, b1_agg, dinv_col, A0, A1, A2, bm1, Wm2, bm2)


def _build_deg(K):
    @functools.partial(
        pl.kernel,
        mesh=_mesh(),
        compiler_params=pltpu.CompilerParams(use_tc_tiling_on_sc=False),
        out_type=(
            jax.ShapeDtypeStruct((N_PAD,), jnp.float32),
            jax.ShapeDtypeStruct((N_PAD,), jnp.float32),
        ),
        scratch_types=[
            pltpu.VMEM((K, CHUNK), jnp.int32),
            pltpu.VMEM((CHUNK,), jnp.float32),
            pltpu.VMEM((NPS,), jnp.float32),
            pltpu.VMEM_SHARED((N_PAD,), jnp.float32),
        ],
    )
    def deg_kernel(dst_hbm, d0_out, d1_out, didx, ones_v, buf, deg_sh):
        c = lax.axis_index("c")
        s = lax.axis_index("s")
        wid = c * NS + s
        pltpu.sync_copy(dst_hbm.at[wid], didx)
        zeros16 = jnp.zeros((LANES,), jnp.float32)

        def zf(i, carry):
            buf[pl.ds(i * LANES, LANES)] = zeros16
            return carry
        lax.fori_loop(0, NPS // LANES, zf, 0)

        def of(i, carry):
            ones_v[pl.ds(i * LANES, LANES)] = zeros16 + 1.0
            return carry
        lax.fori_loop(0, CHUNK // LANES, of, 0)

        pltpu.sync_copy(buf, deg_sh.at[pl.ds(s * NPS, NPS)])
        plsc.subcore_barrier()

        def body(k, carry):
            pltpu.sync_copy(ones_v, deg_sh.at[didx.at[k]], add=True)
            return carry
        lax.fori_loop(0, K, body, 0)

        plsc.subcore_barrier()
        pltpu.sync_copy(deg_sh.at[pl.ds(s * NPS, NPS)], buf)

        @pl.when(c == 0)
        def _():
            pltpu.sync_copy(buf, d0_out.at[pl.ds(s * NPS, NPS)])

        @pl.when(c == 1)
        def _():
            pltpu.sync_copy(buf, d1_out.at[pl.ds(s * NPS, NPS)])

    return deg_kernel


def _build_edges(K):
    KH = K // 2

    @functools.partial(
        pl.kernel,
        mesh=_mesh(),
        compiler_params=pltpu.CompilerParams(use_tc_tiling_on_sc=False),
        out_type=(
            jax.ShapeDtypeStruct((N_PAD, H), jnp.float32),
            jax.ShapeDtypeStruct((N_PAD, H), jnp.float32),
        ),
        scratch_types=[
            pltpu.VMEM((K, CHUNK), jnp.int32),
            pltpu.VMEM((K, CHUNK), jnp.int32),
            pltpu.VMEM((CHUNK, H), jnp.float32),
            pltpu.VMEM((CHUNK, H), jnp.float32),
            pltpu.VMEM_SHARED((N_PAD, H), jnp.float32),
            pltpu.SemaphoreType.DMA,
            pltpu.SemaphoreType.DMA,
        ],
    )
    def edges(q_hbm, src_hbm, dst_hbm, a0_out, a1_out,
              sidx, didx, rows0, rows1, agg_sh, sem0, sem1):
        c = lax.axis_index("c")
        s = lax.axis_index("s")
        wid = c * NS + s
        pltpu.sync_copy(src_hbm.at[wid], sidx)
        pltpu.sync_copy(dst_hbm.at[wid], didx)

        zeros16 = jnp.zeros((LANES,), jnp.float32)

        def zf(i, carry):
            rows0[i, pl.ds(0, LANES)] = zeros16
            rows0[i, pl.ds(LANES, LANES)] = zeros16
            return carry
        lax.fori_loop(0, CHUNK, zf, 0)

        def zs(j, carry):
            pltpu.sync_copy(rows0, agg_sh.at[pl.ds(s * NPS + j * CHUNK, CHUNK)])
            return carry
        lax.fori_loop(0, NPS // CHUNK, zs, 0)

        plsc.subcore_barrier()

        pltpu.async_copy(q_hbm.at[sidx.at[0]], rows0, sem0)

        def body(g, carry):
            k0 = 2 * g
            pltpu.async_copy(q_hbm.at[sidx.at[k0 + 1]], rows1, sem1)
            pltpu.make_async_copy(q_hbm.at[sidx.at[k0]], rows0, sem0).wait()
            pltpu.sync_copy(rows0, agg_sh.at[didx.at[k0]], add=True)

            @pl.when(g + 1 < KH)
            def _():
                pltpu.async_copy(q_hbm.at[sidx.at[k0 + 2]], rows0, sem0)

            pltpu.make_async_copy(q_hbm.at[sidx.at[k0 + 1]], rows1, sem1).wait()
            pltpu.sync_copy(rows1, agg_sh.at[didx.at[k0 + 1]], add=True)
            return carry
        lax.fori_loop(0, KH, body, 0)

        plsc.subcore_barrier()

        @pl.when(c == 0)
        def _():
            def dump(j, carry):
                pltpu.sync_copy(agg_sh.at[pl.ds(s * NPS + j * CHUNK, CHUNK)], rows0)
                pltpu.sync_copy(rows0, a0_out.at[pl.ds(s * NPS + j * CHUNK, CHUNK)])
                return carry
            lax.fori_loop(0, NPS // CHUNK, dump, 0)

        @pl.when(c == 1)
        def _():
            def dump(j, carry):
                pltpu.sync_copy(agg_sh.at[pl.ds(s * NPS + j * CHUNK, CHUNK)], rows0)
                pltpu.sync_copy(rows0, a1_out.at[pl.ds(s * NPS + j * CHUNK, CHUNK)])
                return carry
            lax.fori_loop(0, NPS // CHUNK, dump, 0)

    return edges


def kernel(x, edge_index, W1, b1, W2, b2, Wm1, bm1, Wm2, bm2):
    N = x.shape[0]
    Hh = W1.shape[1]
    E = edge_index.shape[1]
    ept = -(-E // NW)
    K = -(-ept // CHUNK)
    K += K % 2
    E_PAD = K * CHUNK * NW

    src = edge_index[0]
    dst = edge_index[1]
    src_p = jnp.pad(src, (0, E_PAD - E)).reshape(NW, K, CHUNK)
    dst_p = jnp.pad(dst, (0, E_PAD - E), constant_values=N_PAD - 1).reshape(NW, K, CHUNK)
    x_pad = jnp.pad(x, ((0, N_PAD - N), (0, 0)))

    d0, d1 = _build_deg(K)(dst_p)
    h, q0, dinv_col = _tc_mlp(x_pad, W1, b1.reshape(1, -1), W2, b2.reshape(1, -1),
                              d0.reshape(-1, 1), d1.reshape(-1, 1))
    a0, a1 = _build_edges(K)(q0, src_p, dst_p)
    p1, q1 = _tc_combine(h, a0, a1, dinv_col)
    b0, b1_agg = _build_edges(K)(q1, src_p, dst_p)

    Wa, Wb, Wc = Wm1[0:Hh], Wm1[Hh:2 * Hh], Wm1[2 * Hh:3 * Hh]
    A0 = 3.0 * Wa
    A1 = -3.0 * Wa + 3.0 * Wb
    A2 = 0.75 * Wa - 1.5 * Wb + 0.75 * Wc

    out = _tc_head(h, p1, b0, b1_agg, dinv_col, A0, A1, A2,
                   bm1.reshape(1, -1), Wm2, bm2.reshape(1, -1))
    return out[:N]


# speedup vs baseline: 1.9757x; 1.9757x over previous
import functools

import jax
import jax.numpy as jnp
import numpy as np
from jax import lax
from jax.experimental import pallas as pl
from jax.experimental.pallas import tpu as pltpu
from jax.experimental.pallas import tpu_sc as plsc

NC, NS, LANES = 2, 16, 16
NW = NC * NS
N_PAD = 10240
NPT = N_PAD // NW
NPS = N_PAD // NS
H = 32
CHUNK = 128
BM = 2048
_PREC = lax.Precision.HIGHEST


def _mesh():
    return plsc.VectorSubcoreMesh(core_axis_name="c", subcore_axis_name="s")


def _tc_h(x_pad, W1, b1, W2, b2):
    F = x_pad.shape[1]
    Hh = W1.shape[1]

    def body(x_ref, w1_ref, b1_ref, w2_ref, b2_ref, h_ref):
        h1 = jnp.dot(x_ref[...], w1_ref[...],
                     preferred_element_type=jnp.float32, precision=_PREC)
        h1 = jnp.maximum(h1 + b1_ref[...], 0.0)
        h2 = jnp.dot(h1, w2_ref[...],
                     preferred_element_type=jnp.float32, precision=_PREC)
        h_ref[...] = jnp.maximum(h2 + b2_ref[...], 0.0)

    return pl.pallas_call(
        body,
        grid=(N_PAD // BM,),
        in_specs=[
            pl.BlockSpec((BM, F), lambda i: (i, 0)),
            pl.BlockSpec((F, Hh), lambda i: (0, 0)),
            pl.BlockSpec((1, Hh), lambda i: (0, 0)),
            pl.BlockSpec((Hh, Hh), lambda i: (0, 0)),
            pl.BlockSpec((1, Hh), lambda i: (0, 0)),
        ],
        out_specs=pl.BlockSpec((BM, Hh), lambda i: (i, 0)),
        out_shape=jax.ShapeDtypeStruct((N_PAD, Hh), jnp.float32),
    )(x_pad, W1, b1, W2, b2)


def _tc_scale(h, d0_col, d1_col):
    def body(h_ref, d0_ref, d1_ref, q_ref, dinv_ref):
        dinv = lax.rsqrt(jnp.maximum(d0_ref[...] + d1_ref[...], 1.0))
        dinv_ref[...] = dinv
        q_ref[...] = h_ref[...] * dinv

    return pl.pallas_call(
        body,
        grid=(N_PAD // BM,),
        in_specs=[
            pl.BlockSpec((BM, H), lambda i: (i, 0)),
            pl.BlockSpec((BM, 1), lambda i: (i, 0)),
            pl.BlockSpec((BM, 1), lambda i: (i, 0)),
        ],
        out_specs=[
            pl.BlockSpec((BM, H), lambda i: (i, 0)),
            pl.BlockSpec((BM, 1), lambda i: (i, 0)),
        ],
        out_shape=[
            jax.ShapeDtypeStruct((N_PAD, H), jnp.float32),
            jax.ShapeDtypeStruct((N_PAD, 1), jnp.float32),
        ],
    )(h, d0_col, d1_col)


def _tc_combine(h, a0, a1, dinv_col):
    def body(h_ref, a0_ref, a1_ref, dv_ref, p_ref, q_ref):
        dinv = dv_ref[...]
        p = h_ref[...] - dinv * (a0_ref[...] + a1_ref[...])
        p_ref[...] = p
        q_ref[...] = p * dinv

    return pl.pallas_call(
        body,
        grid=(N_PAD // BM,),
        in_specs=[
            pl.BlockSpec((BM, H), lambda i: (i, 0)),
            pl.BlockSpec((BM, H), lambda i: (i, 0)),
            pl.BlockSpec((BM, H), lambda i: (i, 0)),
            pl.BlockSpec((BM, 1), lambda i: (i, 0)),
        ],
        out_specs=[
            pl.BlockSpec((BM, H), lambda i: (i, 0)),
            pl.BlockSpec((BM, H), lambda i: (i, 0)),
        ],
        out_shape=[
            jax.ShapeDtypeStruct((N_PAD, H), jnp.float32),
            jax.ShapeDtypeStruct((N_PAD, H), jnp.float32),
        ],
    )(h, a0, a1, dinv_col)


def _tc_head(h, p1, b0, b1_agg, dinv_col, A0, A1, A2, bm1, Wm2, bm2):
    C = Wm2.shape[1]

    def body(h_ref, p1_ref, b0_ref, b1_ref, dv_ref, a0, a1, a2, b1r, w2r, b2r,
             o_ref):
        p1v = p1_ref[...]
        p2 = p1v - dv_ref[...] * (b0_ref[...] + b1_ref[...])
        t = jnp.dot(h_ref[...], a0[...],
                    preferred_element_type=jnp.float32, precision=_PREC)
        t += jnp.dot(p1v, a1[...],
                     preferred_element_type=jnp.float32, precision=_PREC)
        t += jnp.dot(p2, a2[...],
                     preferred_element_type=jnp.float32, precision=_PREC)
        z = jnp.maximum(t + b1r[...], 0.0)
        o_ref[...] = jnp.dot(z, w2r[...],
                             preferred_element_type=jnp.float32,
                             precision=_PREC) + b2r[...]

    return pl.pallas_call(
        body,
        grid=(N_PAD // BM,),
        in_specs=[
            pl.BlockSpec((BM, H), lambda i: (i, 0)),
            pl.BlockSpec((BM, H), lambda i: (i, 0)),
            pl.BlockSpec((BM, H), lambda i: (i, 0)),
            pl.BlockSpec((BM, H), lambda i: (i, 0)),
            pl.BlockSpec((BM, 1), lambda i: (i, 0)),
            pl.BlockSpec((H, H), lambda i: (0, 0)),
            pl.BlockSpec((H, H), lambda i: (0, 0)),
            pl.BlockSpec((H, H), lambda i: (0, 0)),
            pl.BlockSpec((1, H), lambda i: (0, 0)),
            pl.BlockSpec((H, C), lambda i: (0, 0)),
            pl.BlockSpec((1, C), lambda i: (0, 0)),
        ],
        out_specs=pl.BlockSpec((BM, C), lambda i: (i, 0)),
        out_shape=jax.ShapeDtypeStruct((N_PAD, C), jnp.float32),
    )(h, p1, b0, b1_agg, dinv_col, A0, A1, A2, bm1, Wm2, bm2)


@functools.lru_cache(maxsize=None)
def _build_deg(K):
    @functools.partial(
        pl.kernel,
        mesh=_mesh(),
        compiler_params=pltpu.CompilerParams(use_tc_tiling_on_sc=False),
        out_type=(
            jax.ShapeDtypeStruct((N_PAD,), jnp.float32),
            jax.ShapeDtypeStruct((N_PAD,), jnp.float32),
        ),
        scratch_types=[
            pltpu.VMEM((K, CHUNK), jnp.int32),
            pltpu.VMEM((CHUNK,), jnp.float32),
            pltpu.VMEM((NPS,), jnp.float32),
            pltpu.VMEM_SHARED((N_PAD,), jnp.float32),
            pltpu.SemaphoreType.DMA,
        ],
    )
    def deg_kernel(dst_hbm, d0_out, d1_out, didx, ones_v, buf, deg_sh, dsem):
        c = lax.axis_index("c")
        s = lax.axis_index("s")
        wid = c * NS + s
        pltpu.sync_copy(dst_hbm.at[wid], didx)
        zeros16 = jnp.zeros((LANES,), jnp.float32)

        def zf(i, carry):
            buf[pl.ds(i * LANES, LANES)] = zeros16
            return carry
        lax.fori_loop(0, NPS // LANES, zf, 0)

        def of(i, carry):
            ones_v[pl.ds(i * LANES, LANES)] = zeros16 + 1.0
            return carry
        lax.fori_loop(0, CHUNK // LANES, of, 0)

        pltpu.sync_copy(buf, deg_sh.at[pl.ds(s * NPS, NPS)])
        plsc.subcore_barrier()

        def body(g, carry):
            k0 = 8 * g
            for j in range(8):
                pltpu.async_copy(ones_v, deg_sh.at[didx.at[k0 + j]], dsem,
                                 add=True)
            for j in range(8):
                pltpu.make_async_copy(ones_v, deg_sh.at[didx.at[k0 + j]],
                                      dsem).wait()
            return carry
        lax.fori_loop(0, K // 8, body, 0)

        def tail(k, carry):
            pltpu.sync_copy(ones_v, deg_sh.at[didx.at[k]], add=True)
            return carry
        lax.fori_loop((K // 8) * 8, K, tail, 0)

        plsc.subcore_barrier()
        pltpu.sync_copy(deg_sh.at[pl.ds(s * NPS, NPS)], buf)

        @pl.when(c == 0)
        def _():
            pltpu.sync_copy(buf, d0_out.at[pl.ds(s * NPS, NPS)])

        @pl.when(c == 1)
        def _():
            pltpu.sync_copy(buf, d1_out.at[pl.ds(s * NPS, NPS)])

    return deg_kernel


@functools.lru_cache(maxsize=None)
def _build_edges(K):
    NBUF, G = 8, 4
    assert K % NBUF == 0 and K >= NBUF

    @functools.partial(
        pl.kernel,
        mesh=_mesh(),
        compiler_params=pltpu.CompilerParams(use_tc_tiling_on_sc=False),
        out_type=(
            jax.ShapeDtypeStruct((N_PAD, H), jnp.float32),
            jax.ShapeDtypeStruct((N_PAD, H), jnp.float32),
        ),
        scratch_types=[
            pltpu.VMEM((K, CHUNK), jnp.int32),
            pltpu.VMEM((K, CHUNK), jnp.int32),
            [pltpu.VMEM((CHUNK, H), jnp.float32) for _ in range(NBUF)],
            pltpu.VMEM_SHARED((N_PAD, H), jnp.float32),
            [pltpu.SemaphoreType.DMA for _ in range(NBUF)],
            [pltpu.SemaphoreType.DMA for _ in range(NBUF)],
        ],
    )
    def edges(q_hbm, src_hbm, dst_hbm, a0_out, a1_out,
              sidx, didx, rows, agg_sh, gsems, ssems):
        c = lax.axis_index("c")
        s = lax.axis_index("s")
        wid = c * NS + s
        idx_load = pltpu.async_copy(src_hbm.at[wid], sidx, gsems[0])
        idx_load2 = pltpu.async_copy(dst_hbm.at[wid], didx, gsems[1])

        zeros16 = jnp.zeros((LANES,), jnp.float32)

        def zf(i, carry):
            rows[0][i, pl.ds(0, LANES)] = zeros16
            rows[0][i, pl.ds(LANES, LANES)] = zeros16
            return carry
        lax.fori_loop(0, CHUNK, zf, 0)

        def zs(j, carry):
            pltpu.sync_copy(rows[0], agg_sh.at[pl.ds(s * NPS + j * CHUNK, CHUNK)])
            return carry
        lax.fori_loop(0, NPS // CHUNK, zs, 0)

        idx_load.wait()
        idx_load2.wait()
        plsc.subcore_barrier()

        for j in range(G):
            pltpu.async_copy(q_hbm.at[sidx.at[j]], rows[j], gsems[j])

        def body(g, carry):
            k0 = NBUF * g
            for j in range(NBUF):
                k = k0 + j
                kg = k + G
                jg = (j + G) % NBUF

                @pl.when(kg < K)
                def _():
                    @pl.when(k >= NBUF - G)
                    def _():
                        pltpu.make_async_copy(
                            rows[jg], agg_sh.at[didx.at[kg - NBUF]],
                            ssems[jg]).wait()
                    pltpu.async_copy(q_hbm.at[sidx.at[kg]], rows[jg], gsems[jg])

                pltpu.make_async_copy(q_hbm.at[sidx.at[k]], rows[j], gsems[j]).wait()
                pltpu.async_copy(rows[j], agg_sh.at[didx.at[k]], ssems[j], add=True)
            return carry
        lax.fori_loop(0, K // NBUF, body, 0)

        for j in range(NBUF):
            pltpu.make_async_copy(rows[j], agg_sh.at[didx.at[K - NBUF + j]],
                                  ssems[j]).wait()

        plsc.subcore_barrier()

        @pl.when(c == 0)
        def _():
            def dump(j, carry):
                pltpu.sync_copy(agg_sh.at[pl.ds(s * NPS + j * CHUNK, CHUNK)], rows[0])
                pltpu.sync_copy(rows[0], a0_out.at[pl.ds(s * NPS + j * CHUNK, CHUNK)])
                return carry
            lax.fori_loop(0, NPS // CHUNK, dump, 0)

        @pl.when(c == 1)
        def _():
            def dump(j, carry):
                pltpu.sync_copy(agg_sh.at[pl.ds(s * NPS + j * CHUNK, CHUNK)], rows[0])
                pltpu.sync_copy(rows[0], a1_out.at[pl.ds(s * NPS + j * CHUNK, CHUNK)])
                return carry
            lax.fori_loop(0, NPS // CHUNK, dump, 0)

    return edges


def kernel(x, edge_index, W1, b1, W2, b2, Wm1, bm1, Wm2, bm2):
    N = x.shape[0]
    Hh = W1.shape[1]
    E = edge_index.shape[1]
    ept = -(-E // NW)
    K = -(-ept // CHUNK)
    K = -(-K // 4) * 4
    E_PAD = K * CHUNK * NW

    src = edge_index[0]
    dst = edge_index[1]
    ept_real = -(-E // NW)
    if E != ept_real * NW:
        flat_pad = ept_real * NW - E
        src = jnp.pad(src, (0, flat_pad))
        dst = jnp.concatenate(
            [dst, jnp.asarray(N + (np.arange(flat_pad) % (N_PAD - N)),
                              dtype=jnp.int32)])
    pad_per_tile = K * CHUNK - ept_real
    pad_rows = np.broadcast_to(
        N + (np.arange(pad_per_tile, dtype=np.int32) % (N_PAD - N)),
        (NW, pad_per_tile))
    pad_src = np.broadcast_to(
        np.arange(pad_per_tile, dtype=np.int32) % N, (NW, pad_per_tile))
    src_p = jnp.concatenate(
        [src.reshape(NW, ept_real), jnp.asarray(pad_src)], axis=1
    ).reshape(NW, K, CHUNK)
    dst_p = jnp.concatenate(
        [dst.reshape(NW, ept_real), jnp.asarray(pad_rows)], axis=1
    ).reshape(NW, K, CHUNK)
    x_pad = jnp.pad(x, ((0, N_PAD - N), (0, 0)))

    d0, d1 = _build_deg(K)(dst_p)
    h = _tc_h(x_pad, W1, b1.reshape(1, -1), W2, b2.reshape(1, -1))
    q0, dinv_col = _tc_scale(h, d0.reshape(-1, 1), d1.reshape(-1, 1))
    a0, a1 = _build_edges(K)(q0, src_p, dst_p)
    p1, q1 = _tc_combine(h, a0, a1, dinv_col)
    b0, b1_agg = _build_edges(K)(q1, src_p, dst_p)

    Wa, Wb, Wc = Wm1[0:Hh], Wm1[Hh:2 * Hh], Wm1[2 * Hh:3 * Hh]
    A0 = 3.0 * Wa
    A1 = -3.0 * Wa + 3.0 * Wb
    A2 = 0.75 * Wa - 1.5 * Wb + 0.75 * Wc

    out = _tc_head(h, p1, b0, b1_agg, dinv_col, A0, A1, A2,
                   bm1.reshape(1, -1), Wm2, bm2.reshape(1, -1))
    return out[:N]
